# baseline (speedup 1.0000x reference)
import jax
import jax.numpy as jnp
from jax.experimental import pallas as pl
from jax.experimental.pallas import tpu as pltpu

Q = 1024
D = 128
K = 100000
BK = 5000
NKB = K // BK


def _knn_body(am2_ref, p2_ref, p2s_ref, cols_ref, inds_ref, dis_ref,
              p1sq_ref, mina_ref, idxa_ref, minb_ref, idxb_ref):
    k = pl.program_id(0)

    @pl.when(k == 0)
    def _init():
        am2 = am2_ref[...]
        p1sq_ref[...] = 0.25 * jnp.sum(am2 * am2, axis=1, keepdims=True)
        mina_ref[...] = jnp.full((Q, 1), jnp.inf, jnp.float32)
        minb_ref[...] = jnp.full((Q, 1), jnp.inf, jnp.float32)
        idxa_ref[...] = jnp.zeros((Q, 1), jnp.float32)
        idxb_ref[...] = jnp.zeros((Q, 1), jnp.float32)

    am2 = am2_ref[...]
    p1sq = p1sq_ref[...]
    cols = cols_ref[...]
    base = (k * BK).astype(jnp.float32)

    def block_min(b):
        dot2 = jax.lax.dot_general(
            am2, b, (((1,), (1,)), ((), ())),
            preferred_element_type=jnp.float32)
        p2sq = jnp.sum(b * b, axis=1)[None, :]
        d2 = (p1sq + dot2) + p2sq
        bmin = jnp.min(d2, axis=1, keepdims=True)
        mask = jnp.where(d2 == bmin, jnp.float32(1),
                         jnp.float32(0)).astype(jnp.bfloat16)
        s = jax.lax.dot_general(
            mask, cols, (((1,), (0,)), ((), ())),
            preferred_element_type=jnp.float32)
        lidx = s[:, 0:1] + s[:, 1:2]
        cnt = s[:, 2:3]
        return d2, bmin, lidx, cnt

    d2a, bmin_a, lidx_a, cnt_a = block_min(p2_ref[...])
    d2b, bmin_b, lidx_b, cnt_b = block_min(p2s_ref[...])
    tie = jnp.logical_or(jnp.max(cnt_a) > 1.5, jnp.max(cnt_b) > 1.5)

    def upd(min_ref, idx_ref, bmin, bidx):
        hit = bmin < min_ref[...]
        min_ref[...] = jnp.where(hit, bmin, min_ref[...])
        idx_ref[...] = jnp.where(hit, bidx, idx_ref[...])

    @pl.when(jnp.logical_not(tie))
    def _fast():
        upd(mina_ref, idxa_ref, bmin_a, base + lidx_a)
        upd(minb_ref, idxb_ref, bmin_b, base + lidx_b)

    @pl.when(tie)
    def _slow():
        iota_f = jax.lax.broadcasted_iota(
            jnp.int32, (Q, BK), 1).astype(jnp.float32)
        la = jnp.min(jnp.where(d2a == bmin_a, iota_f, jnp.float32(3e38)),
                     axis=1, keepdims=True)
        lb = jnp.min(jnp.where(d2b == bmin_b, iota_f, jnp.float32(3e38)),
                     axis=1, keepdims=True)
        upd(mina_ref, idxa_ref, bmin_a, base + la)
        upd(minb_ref, idxb_ref, bmin_b, base + lb)

    @pl.when(k == NKB - 1)
    def _fin():
        sym_mask = mina_ref[...] < minb_ref[...]
        inds_ref[...] = jnp.where(sym_mask, idxa_ref[...],
                                  idxb_ref[...]).astype(jnp.int32)
        dis_ref[...] = jnp.where(sym_mask, mina_ref[...], minb_ref[...])


@jax.jit
def _run(am2, p2, p2s):
    j = jnp.arange(BK, dtype=jnp.int32)
    cols = jnp.zeros((BK, 128), jnp.bfloat16)
    cols = cols.at[:, 0].set(((j // 256) * 256).astype(jnp.bfloat16))
    cols = cols.at[:, 1].set((j % 256).astype(jnp.bfloat16))
    cols = cols.at[:, 2].set(jnp.bfloat16(1))
    return pl.pallas_call(
        _knn_body,
        grid=(NKB,),
        in_specs=[
            pl.BlockSpec((Q, D), lambda k: (0, 0)),
            pl.BlockSpec((BK, D), lambda k: (k, 0)),
            pl.BlockSpec((BK, D), lambda k: (k, 0)),
            pl.BlockSpec((BK, 128), lambda k: (0, 0)),
        ],
        out_specs=[
            pl.BlockSpec((Q, 1), lambda k: (0, 0)),
            pl.BlockSpec((Q, 1), lambda k: (0, 0)),
        ],
        out_shape=[
            jax.ShapeDtypeStruct((Q, 1), jnp.int32),
            jax.ShapeDtypeStruct((Q, 1), jnp.float32),
        ],
        scratch_shapes=[
            pltpu.VMEM((Q, 1), jnp.float32),
            pltpu.VMEM((Q, 1), jnp.float32),
            pltpu.VMEM((Q, 1), jnp.float32),
            pltpu.VMEM((Q, 1), jnp.float32),
            pltpu.VMEM((Q, 1), jnp.float32),
        ],
    )(am2, p2, p2s, cols)


def kernel(p1_key_points, p2_key_points, p2_key_points_sym):
    am2 = -2.0 * p1_key_points[0]
    inds, dis = _run(am2, p2_key_points[0], p2_key_points_sym[0])
    return inds[None].astype(jnp.int64), dis[None]


# speedup vs baseline: 1.6832x; 1.6832x over previous
import jax
import jax.numpy as jnp
from jax.experimental import pallas as pl
from jax.experimental.pallas import tpu as pltpu

Q = 1024
D = 128
K = 100000
BK = 10000
NKB = K // BK
CH = 2000
NCH = BK // CH


def _knn_body(am2_ref, p2_ref, p2s_ref, inds_ref, dis_ref,
              p1sq_ref, mina_ref, idxa_ref, minb_ref, idxb_ref):
    k = pl.program_id(0)

    @pl.when(k == 0)
    def _init():
        am2 = am2_ref[...]
        p1sq_ref[...] = 0.25 * jnp.sum(am2 * am2, axis=1, keepdims=True)
        mina_ref[...] = jnp.full((Q, 1), jnp.inf, jnp.float32)
        minb_ref[...] = jnp.full((Q, 1), jnp.inf, jnp.float32)
        idxa_ref[...] = jnp.zeros((Q, 1), jnp.float32)
        idxb_ref[...] = jnp.zeros((Q, 1), jnp.float32)

    am2 = am2_ref[...]
    p1sq = p1sq_ref[...]
    iota_f = jax.lax.broadcasted_iota(jnp.int32, (Q, CH), 1).astype(jnp.float32)
    base = (k * BK).astype(jnp.float32)

    for b_ref, min_ref, idx_ref in ((p2_ref, mina_ref, idxa_ref),
                                    (p2s_ref, minb_ref, idxb_ref)):
        for c in range(NCH):
            b = b_ref[c * CH:(c + 1) * CH, :]
            dot2 = jax.lax.dot_general(
                am2, b, (((1,), (1,)), ((), ())),
                preferred_element_type=jnp.float32)
            p2sq = jnp.sum(b * b, axis=1)[None, :]
            d2 = (p1sq + dot2) + p2sq
            bmin = jnp.min(d2, axis=1, keepdims=True)
            lidx = jnp.min(jnp.where(d2 == bmin, iota_f, jnp.float32(3e38)),
                           axis=1, keepdims=True)
            cand = (base + jnp.float32(c * CH)) + lidx
            hit = bmin < min_ref[...]
            min_ref[...] = jnp.where(hit, bmin, min_ref[...])
            idx_ref[...] = jnp.where(hit, cand, idx_ref[...])

    @pl.when(k == NKB - 1)
    def _fin():
        sym_mask = mina_ref[...] < minb_ref[...]
        inds_ref[...] = jnp.where(sym_mask, idxa_ref[...],
                                  idxb_ref[...]).astype(jnp.int32)
        dis_ref[...] = jnp.where(sym_mask, mina_ref[...], minb_ref[...])


@jax.jit
def _run(am2, p2, p2s):
    return pl.pallas_call(
        _knn_body,
        grid=(NKB,),
        in_specs=[
            pl.BlockSpec((Q, D), lambda k: (0, 0)),
            pl.BlockSpec((BK, D), lambda k: (k, 0)),
            pl.BlockSpec((BK, D), lambda k: (k, 0)),
        ],
        out_specs=[
            pl.BlockSpec((Q, 1), lambda k: (0, 0)),
            pl.BlockSpec((Q, 1), lambda k: (0, 0)),
        ],
        out_shape=[
            jax.ShapeDtypeStruct((Q, 1), jnp.int32),
            jax.ShapeDtypeStruct((Q, 1), jnp.float32),
        ],
        scratch_shapes=[
            pltpu.VMEM((Q, 1), jnp.float32),
            pltpu.VMEM((Q, 1), jnp.float32),
            pltpu.VMEM((Q, 1), jnp.float32),
            pltpu.VMEM((Q, 1), jnp.float32),
            pltpu.VMEM((Q, 1), jnp.float32),
        ],
    )(am2, p2, p2s)


def kernel(p1_key_points, p2_key_points, p2_key_points_sym):
    am2 = -2.0 * p1_key_points[0]
    inds, dis = _run(am2, p2_key_points[0], p2_key_points_sym[0])
    return inds[None].astype(jnp.int64), dis[None]
